# baseline (speedup 1.0000x reference)
import functools

import jax
import jax.numpy as jnp
from jax.experimental import pallas as pl

_HC = 128
_COND = 64
_NR = 6
_NSR = 42
_BE = 8
_IE = 64
_OE = 256
_NB = 4
_NN = 10000
_NE = 160000
_NT = 640000

_NODE_BLK = 2000
_EDGE_BLK = 4000
_TRIP_BLK = 8000


def _silu(t):
    return t * jax.nn.sigmoid(t)


def _full(shape):
    return pl.BlockSpec(shape, lambda i: (0,) * len(shape))


def _node_embed_body(z_ref, cond_ref, emb_ref, w1_ref, w2_ref, ha_ref, hb_ref):
    z = z_ref[...]
    oh = (z == jax.lax.broadcasted_iota(jnp.int32, (z.shape[0], 95), 1)).astype(jnp.float32)
    h = jnp.dot(oh, emb_ref[...], preferred_element_type=jnp.float32)
    hc = jnp.concatenate([h, cond_ref[...]], axis=1)
    ha_ref[...] = jnp.dot(hc, w1_ref[...], preferred_element_type=jnp.float32)
    hb_ref[...] = jnp.dot(hc, w2_ref[...], preferred_element_type=jnp.float32)


def _node_embed(z2, cond, emb, w1, w2):
    B = _NODE_BLK
    grid = _NN // B
    return pl.pallas_call(
        _node_embed_body,
        grid=(grid,),
        in_specs=[
            pl.BlockSpec((B, 1), lambda i: (i, 0)),
            pl.BlockSpec((B, _COND), lambda i: (i, 0)),
            _full(emb.shape), _full(w1.shape), _full(w2.shape),
        ],
        out_specs=[
            pl.BlockSpec((B, _HC), lambda i: (i, 0)),
            pl.BlockSpec((B, _HC), lambda i: (i, 0)),
        ],
        out_shape=[
            jax.ShapeDtypeStruct((_NN, _HC), jnp.float32),
            jax.ShapeDtypeStruct((_NN, _HC), jnp.float32),
        ],
    )(z2, cond, emb, w1, w2)


def _edge_init_body(ga_ref, gb_ref, rbf_ref, wr_ref, br_ref, w3_ref, b3_ref,
                    orbf_ref, x_ref, g0_ref):
    r = _silu(jnp.dot(rbf_ref[...], wr_ref[...], preferred_element_type=jnp.float32) + br_ref[...])
    x = _silu(ga_ref[...] + gb_ref[...]
              + jnp.dot(r, w3_ref[...], preferred_element_type=jnp.float32) + b3_ref[...])
    x_ref[...] = x
    g0_ref[...] = jnp.dot(rbf_ref[...], orbf_ref[...], preferred_element_type=jnp.float32) * x


def _edge_init(ga, gb, rbf, wr, br, w3, b3, orbf):
    B = _EDGE_BLK
    grid = _NE // B
    return pl.pallas_call(
        _edge_init_body,
        grid=(grid,),
        in_specs=[
            pl.BlockSpec((B, _HC), lambda i: (i, 0)),
            pl.BlockSpec((B, _HC), lambda i: (i, 0)),
            pl.BlockSpec((B, _NR), lambda i: (i, 0)),
            _full(wr.shape), _full(br.shape), _full(w3.shape), _full(b3.shape),
            _full(orbf.shape),
        ],
        out_specs=[
            pl.BlockSpec((B, _HC), lambda i: (i, 0)),
            pl.BlockSpec((B, _HC), lambda i: (i, 0)),
        ],
        out_shape=[
            jax.ShapeDtypeStruct((_NE, _HC), jnp.float32),
            jax.ShapeDtypeStruct((_NE, _HC), jnp.float32),
        ],
    )(ga, gb, rbf, wr, br, w3, b3, orbf)


def _pre_body(x_ref, rbf_ref, jiw, jib, kjw, kjb, r1, r2, dw, xji_ref, xdown_ref):
    x = x_ref[...]
    xji_ref[...] = _silu(jnp.dot(x, jiw[...], preferred_element_type=jnp.float32) + jib[...])
    xkj = _silu(jnp.dot(x, kjw[...], preferred_element_type=jnp.float32) + kjb[...])
    r = jnp.dot(jnp.dot(rbf_ref[...], r1[...], preferred_element_type=jnp.float32),
                r2[...], preferred_element_type=jnp.float32)
    xdown_ref[...] = _silu(jnp.dot(xkj * r, dw[...], preferred_element_type=jnp.float32))


def _pre(x, rbf, b):
    B = _EDGE_BLK
    grid = _NE // B
    return pl.pallas_call(
        _pre_body,
        grid=(grid,),
        in_specs=[
            pl.BlockSpec((B, _HC), lambda i: (i, 0)),
            pl.BlockSpec((B, _NR), lambda i: (i, 0)),
            _full((_HC, _HC)), _full((_HC,)), _full((_HC, _HC)), _full((_HC,)),
            _full((_NR, _BE)), _full((_BE, _HC)), _full((_HC, _IE)),
        ],
        out_specs=[
            pl.BlockSpec((B, _HC), lambda i: (i, 0)),
            pl.BlockSpec((B, _IE), lambda i: (i, 0)),
        ],
        out_shape=[
            jax.ShapeDtypeStruct((_NE, _HC), jnp.float32),
            jax.ShapeDtypeStruct((_NE, _IE), jnp.float32),
        ],
    )(x, rbf, b["ji_w"], b["ji_b"], b["kj_w"], b["kj_b"], b["rbf1"], b["rbf2"], b["down"])


def _mm_body(a_ref, w_ref, o_ref):
    o_ref[...] = jnp.dot(a_ref[...], w_ref[...], preferred_element_type=jnp.float32)


def _mm(a, w, blk):
    n, k = a.shape
    m = w.shape[1]
    return pl.pallas_call(
        _mm_body,
        grid=(n // blk,),
        in_specs=[pl.BlockSpec((blk, k), lambda i: (i, 0)), _full(w.shape)],
        out_specs=pl.BlockSpec((blk, m), lambda i: (i, 0)),
        out_shape=jax.ShapeDtypeStruct((n, m), jnp.float32),
    )(a, w)


def _post_body(acc_ref, xji_ref, x_ref, rbf_ref, upw, bw1, bb1, bw2, bb2,
               linw, linb, aw1, ab1, aw2, ab2, cw1, cb1, cw2, cb2, orbf,
               xnew_ref, g_ref):
    def dot(a, w):
        return jnp.dot(a, w[...], preferred_element_type=jnp.float32)

    u = _silu(dot(acc_ref[...], upw))
    h = xji_ref[...] + u
    h = h + _silu(dot(_silu(dot(h, bw1) + bb1[...]), bw2) + bb2[...])
    h = _silu(dot(h, linw) + linb[...]) + x_ref[...]
    h = h + _silu(dot(_silu(dot(h, aw1) + ab1[...]), aw2) + ab2[...])
    h = h + _silu(dot(_silu(dot(h, cw1) + cb1[...]), cw2) + cb2[...])
    xnew_ref[...] = h
    g_ref[...] = dot(rbf_ref[...], orbf) * h


def _post(acc, xji, x, rbf, b, orbf):
    B = _EDGE_BLK
    grid = _NE // B
    r0, r1, r2 = b["before"][0], b["after"][0], b["after"][1]
    return pl.pallas_call(
        _post_body,
        grid=(grid,),
        in_specs=[
            pl.BlockSpec((B, _IE), lambda i: (i, 0)),
            pl.BlockSpec((B, _HC), lambda i: (i, 0)),
            pl.BlockSpec((B, _HC), lambda i: (i, 0)),
            pl.BlockSpec((B, _NR), lambda i: (i, 0)),
            _full((_IE, _HC)),
            _full((_HC, _HC)), _full((_HC,)), _full((_HC, _HC)), _full((_HC,)),
            _full((_HC, _HC)), _full((_HC,)),
            _full((_HC, _HC)), _full((_HC,)), _full((_HC, _HC)), _full((_HC,)),
            _full((_HC, _HC)), _full((_HC,)), _full((_HC, _HC)), _full((_HC,)),
            _full((_NR, _HC)),
        ],
        out_specs=[
            pl.BlockSpec((B, _HC), lambda i: (i, 0)),
            pl.BlockSpec((B, _HC), lambda i: (i, 0)),
        ],
        out_shape=[
            jax.ShapeDtypeStruct((_NE, _HC), jnp.float32),
            jax.ShapeDtypeStruct((_NE, _HC), jnp.float32),
        ],
    )(acc, xji, x, rbf,
      b["up"], r0["w1"], r0["b1"], r0["w2"], r0["b2"], b["lin_w"], b["lin_b"],
      r1["w1"], r1["b1"], r1["w2"], r1["b2"], r2["w1"], r2["b1"], r2["w2"], r2["b2"],
      orbf)


def _out_body(gs_ref, upw, upb, lw, lb, linw, p_ref):
    p = jnp.zeros((gs_ref.shape[1], 1), jnp.float32)
    for b in range(_NB + 1):
        y = jnp.dot(gs_ref[b], upw[b], preferred_element_type=jnp.float32) + upb[b]
        for l in range(3):
            y = _silu(jnp.dot(y, lw[b, l], preferred_element_type=jnp.float32) + lb[b, l])
        p = p + jnp.dot(y, linw[b], preferred_element_type=jnp.float32)
    p_ref[...] = p


def _out_mlps(gs, upw, upb, lw, lb, linw):
    B = _NODE_BLK
    grid = _NN // B
    return pl.pallas_call(
        _out_body,
        grid=(grid,),
        in_specs=[
            pl.BlockSpec((_NB + 1, B, _HC), lambda i: (0, i, 0)),
            _full(upw.shape), _full(upb.shape), _full(lw.shape), _full(lb.shape),
            _full(linw.shape),
        ],
        out_specs=pl.BlockSpec((B, 1), lambda i: (i, 0)),
        out_shape=jax.ShapeDtypeStruct((_NN, 1), jnp.float32),
    )(gs, upw, upb, lw, lb, linw)


def kernel(params, z, cond, rbf, sbf, i, j, idx_kj, idx_ji):
    w1 = params["emb_lin_w"][: _HC + _COND]
    w2 = params["emb_lin_w"][_HC + _COND: 2 * (_HC + _COND)]
    w3 = params["emb_lin_w"][2 * (_HC + _COND):]

    ha, hb = _node_embed(z.astype(jnp.int32).reshape(_NN, 1), cond, params["emb"], w1, w2)

    ga = jnp.take(ha, i, axis=0)
    gb = jnp.take(hb, j, axis=0)
    x, g0 = _edge_init(ga, gb, rbf, params["emb_rbf_w"], params["emb_rbf_b"],
                       w3, params["emb_lin_b"], params["outs"][0]["rbf"])

    sbf1_all = jnp.concatenate([b["sbf1"] for b in params["blocks"]], axis=1)
    t_all = _mm(sbf, sbf1_all, _TRIP_BLK)

    gs = [jax.ops.segment_sum(g0, i, num_segments=_NN)]
    for bi in range(_NB):
        b = params["blocks"][bi]
        xji, xdown = _pre(x, rbf, b)
        s = _mm(t_all[:, bi * _BE:(bi + 1) * _BE], b["sbf2"], _TRIP_BLK)
        m = jnp.take(xdown, idx_kj, axis=0) * s
        acc = jax.ops.segment_sum(m, idx_ji, num_segments=_NE)
        x, g = _post(acc, xji, x, rbf, b, params["outs"][bi + 1]["rbf"])
        gs.append(jax.ops.segment_sum(g, i, num_segments=_NN))

    upw = jnp.stack([o["up_w"] for o in params["outs"]])
    upb = jnp.stack([o["up_b"] for o in params["outs"]])
    lw = jnp.stack([jnp.stack([l["w"] for l in o["lins"]]) for o in params["outs"]])
    lb = jnp.stack([jnp.stack([l["b"] for l in o["lins"]]) for o in params["outs"]])
    linw = jnp.stack([o["lin"] for o in params["outs"]])
    return _out_mlps(jnp.stack(gs), upw, upb, lw, lb, linw)


# speedup vs baseline: 2.4519x; 2.4519x over previous
import functools

import jax
import jax.numpy as jnp
from jax.experimental import pallas as pl
from jax.experimental.pallas import tpu as pltpu
from jax.experimental.pallas import tpu_sc as plsc

_HC = 128
_COND = 64
_NR = 6
_NSR = 42
_BE = 8
_IE = 64
_OE = 256
_NB = 4
_NN = 10000
_NE = 160000
_NT = 640000

_NODE_BLK = 2000
_EDGE_BLK = 4000
_TRIP_BLK = 8000


def _silu(t):
    return t * jax.nn.sigmoid(t)


def _full(shape):
    return pl.BlockSpec(shape, lambda i: (0,) * len(shape))


def _node_embed_body(z_ref, cond_ref, emb_ref, w1_ref, w2_ref, ha_ref, hb_ref):
    z = z_ref[...]
    oh = (z == jax.lax.broadcasted_iota(jnp.int32, (z.shape[0], 95), 1)).astype(jnp.float32)
    h = jnp.dot(oh, emb_ref[...], preferred_element_type=jnp.float32)
    hc = jnp.concatenate([h, cond_ref[...]], axis=1)
    ha_ref[...] = jnp.dot(hc, w1_ref[...], preferred_element_type=jnp.float32)
    hb_ref[...] = jnp.dot(hc, w2_ref[...], preferred_element_type=jnp.float32)


def _node_embed(z2, cond, emb, w1, w2):
    B = _NODE_BLK
    grid = _NN // B
    return pl.pallas_call(
        _node_embed_body,
        grid=(grid,),
        in_specs=[
            pl.BlockSpec((B, 1), lambda i: (i, 0)),
            pl.BlockSpec((B, _COND), lambda i: (i, 0)),
            _full(emb.shape), _full(w1.shape), _full(w2.shape),
        ],
        out_specs=[
            pl.BlockSpec((B, _HC), lambda i: (i, 0)),
            pl.BlockSpec((B, _HC), lambda i: (i, 0)),
        ],
        out_shape=[
            jax.ShapeDtypeStruct((_NN, _HC), jnp.float32),
            jax.ShapeDtypeStruct((_NN, _HC), jnp.float32),
        ],
    )(z2, cond, emb, w1, w2)


def _edge_init_body(ga_ref, gb_ref, rbf_ref, wr_ref, br_ref, w3_ref, b3_ref,
                    orbf_ref, x_ref, g0_ref):
    r = _silu(jnp.dot(rbf_ref[...], wr_ref[...], preferred_element_type=jnp.float32) + br_ref[...])
    x = _silu(ga_ref[...] + gb_ref[...]
              + jnp.dot(r, w3_ref[...], preferred_element_type=jnp.float32) + b3_ref[...])
    x_ref[...] = x
    g0_ref[...] = jnp.dot(rbf_ref[...], orbf_ref[...], preferred_element_type=jnp.float32) * x


def _edge_init(ga, gb, rbf, wr, br, w3, b3, orbf):
    B = _EDGE_BLK
    grid = _NE // B
    return pl.pallas_call(
        _edge_init_body,
        grid=(grid,),
        in_specs=[
            pl.BlockSpec((B, _HC), lambda i: (i, 0)),
            pl.BlockSpec((B, _HC), lambda i: (i, 0)),
            pl.BlockSpec((B, _NR), lambda i: (i, 0)),
            _full(wr.shape), _full(br.shape), _full(w3.shape), _full(b3.shape),
            _full(orbf.shape),
        ],
        out_specs=[
            pl.BlockSpec((B, _HC), lambda i: (i, 0)),
            pl.BlockSpec((B, _HC), lambda i: (i, 0)),
        ],
        out_shape=[
            jax.ShapeDtypeStruct((_NE, _HC), jnp.float32),
            jax.ShapeDtypeStruct((_NE, _HC), jnp.float32),
        ],
    )(ga, gb, rbf, wr, br, w3, b3, orbf)


def _pre_body(x_ref, rbf_ref, jiw, jib, kjw, kjb, r1, r2, dw, xji_ref, xdown_ref):
    x = x_ref[...]
    xji_ref[...] = _silu(jnp.dot(x, jiw[...], preferred_element_type=jnp.float32) + jib[...])
    xkj = _silu(jnp.dot(x, kjw[...], preferred_element_type=jnp.float32) + kjb[...])
    r = jnp.dot(jnp.dot(rbf_ref[...], r1[...], preferred_element_type=jnp.float32),
                r2[...], preferred_element_type=jnp.float32)
    xdown_ref[...] = _silu(jnp.dot(xkj * r, dw[...], preferred_element_type=jnp.float32))


def _pre(x, rbf, b):
    B = _EDGE_BLK
    grid = _NE // B
    return pl.pallas_call(
        _pre_body,
        grid=(grid,),
        in_specs=[
            pl.BlockSpec((B, _HC), lambda i: (i, 0)),
            pl.BlockSpec((B, _NR), lambda i: (i, 0)),
            _full((_HC, _HC)), _full((_HC,)), _full((_HC, _HC)), _full((_HC,)),
            _full((_NR, _BE)), _full((_BE, _HC)), _full((_HC, _IE)),
        ],
        out_specs=[
            pl.BlockSpec((B, _HC), lambda i: (i, 0)),
            pl.BlockSpec((B, _IE), lambda i: (i, 0)),
        ],
        out_shape=[
            jax.ShapeDtypeStruct((_NE, _HC), jnp.float32),
            jax.ShapeDtypeStruct((_NE, _IE), jnp.float32),
        ],
    )(x, rbf, b["ji_w"], b["ji_b"], b["kj_w"], b["kj_b"], b["rbf1"], b["rbf2"], b["down"])


def _mm_body(a_ref, w_ref, o_ref):
    o_ref[...] = jnp.dot(a_ref[...], w_ref[...], preferred_element_type=jnp.float32)


def _mm(a, w, blk):
    n, k = a.shape
    m = w.shape[1]
    return pl.pallas_call(
        _mm_body,
        grid=(n // blk,),
        in_specs=[pl.BlockSpec((blk, k), lambda i: (i, 0)), _full(w.shape)],
        out_specs=pl.BlockSpec((blk, m), lambda i: (i, 0)),
        out_shape=jax.ShapeDtypeStruct((n, m), jnp.float32),
    )(a, w)


def _post_body(acc_ref, xji_ref, x_ref, rbf_ref, upw, bw1, bb1, bw2, bb2,
               linw, linb, aw1, ab1, aw2, ab2, cw1, cb1, cw2, cb2, orbf,
               xnew_ref, g_ref):
    def dot(a, w):
        return jnp.dot(a, w[...], preferred_element_type=jnp.float32)

    u = _silu(dot(acc_ref[...], upw))
    h = xji_ref[...] + u
    h = h + _silu(dot(_silu(dot(h, bw1) + bb1[...]), bw2) + bb2[...])
    h = _silu(dot(h, linw) + linb[...]) + x_ref[...]
    h = h + _silu(dot(_silu(dot(h, aw1) + ab1[...]), aw2) + ab2[...])
    h = h + _silu(dot(_silu(dot(h, cw1) + cb1[...]), cw2) + cb2[...])
    xnew_ref[...] = h
    g_ref[...] = dot(rbf_ref[...], orbf) * h


def _post(acc, xji, x, rbf, b, orbf):
    B = _EDGE_BLK
    grid = _NE // B
    r0, r1, r2 = b["before"][0], b["after"][0], b["after"][1]
    return pl.pallas_call(
        _post_body,
        grid=(grid,),
        in_specs=[
            pl.BlockSpec((B, _IE), lambda i: (i, 0)),
            pl.BlockSpec((B, _HC), lambda i: (i, 0)),
            pl.BlockSpec((B, _HC), lambda i: (i, 0)),
            pl.BlockSpec((B, _NR), lambda i: (i, 0)),
            _full((_IE, _HC)),
            _full((_HC, _HC)), _full((_HC,)), _full((_HC, _HC)), _full((_HC,)),
            _full((_HC, _HC)), _full((_HC,)),
            _full((_HC, _HC)), _full((_HC,)), _full((_HC, _HC)), _full((_HC,)),
            _full((_HC, _HC)), _full((_HC,)), _full((_HC, _HC)), _full((_HC,)),
            _full((_NR, _HC)),
        ],
        out_specs=[
            pl.BlockSpec((B, _HC), lambda i: (i, 0)),
            pl.BlockSpec((B, _HC), lambda i: (i, 0)),
        ],
        out_shape=[
            jax.ShapeDtypeStruct((_NE, _HC), jnp.float32),
            jax.ShapeDtypeStruct((_NE, _HC), jnp.float32),
        ],
    )(acc, xji, x, rbf,
      b["up"], r0["w1"], r0["b1"], r0["w2"], r0["b2"], b["lin_w"], b["lin_b"],
      r1["w1"], r1["b1"], r1["w2"], r1["b2"], r2["w1"], r2["b1"], r2["w2"], r2["b2"],
      orbf)


def _out_body(gs_ref, upw, upb, lw, lb, linw, p_ref):
    p = jnp.zeros((gs_ref.shape[1], 1), jnp.float32)
    for b in range(_NB + 1):
        y = jnp.dot(gs_ref[b], upw[b], preferred_element_type=jnp.float32) + upb[b]
        for l in range(3):
            y = _silu(jnp.dot(y, lw[b, l], preferred_element_type=jnp.float32) + lb[b, l])
        p = p + jnp.dot(y, linw[b], preferred_element_type=jnp.float32)
    p_ref[...] = p


def _out_mlps(gs, upw, upb, lw, lb, linw):
    B = _NODE_BLK
    grid = _NN // B
    return pl.pallas_call(
        _out_body,
        grid=(grid,),
        in_specs=[
            pl.BlockSpec((_NB + 1, B, _HC), lambda i: (0, i, 0)),
            _full(upw.shape), _full(upb.shape), _full(lw.shape), _full(lb.shape),
            _full(linw.shape),
        ],
        out_specs=pl.BlockSpec((B, 1), lambda i: (i, 0)),
        out_shape=jax.ShapeDtypeStruct((_NN, 1), jnp.float32),
    )(gs, upw, upb, lw, lb, linw)


_W = 1000
_TPT = _NT // 16
_EPT = _NE // 16


def _trip_body(xd_ref, s_ref, ikj_ref, iji_ref, zero_ref, out_ref,
               ikj_v, iji_v, s_v, rows_v, zbuf, acc_sh, sem):
    core = jax.lax.axis_index("c")
    sid = jax.lax.axis_index("s")
    pltpu.sync_copy(zero_ref, zbuf)
    for cc in range(4):
        chunk = core * 4 + cc
        for kz in range(_EPT // _W):
            pltpu.sync_copy(zbuf, acc_sh.at[pl.ds(sid * _EPT + kz * _W, _W)])
        plsc.subcore_barrier()

        def win(w, _):
            base = sid * _TPT + w * _W
            pltpu.sync_copy(ikj_ref.at[pl.ds(base, _W)], ikj_v)
            pltpu.sync_copy(iji_ref.at[pl.ds(base, _W)], iji_v)
            pltpu.sync_copy(s_ref.at[chunk].at[pl.ds(base, _W)], s_v)
            pltpu.async_copy(xd_ref.at[chunk].at[ikj_v], rows_v, sem).wait()
            def mul(k, carry):
                lanes = jax.lax.iota(jnp.int32, 16)
                brow = lanes // 8
                bcol = lanes % 8
                ridx = brow + 2 * k
                va = plsc.load_gather(rows_v, [ridx, bcol])
                vb = plsc.load_gather(s_v, [ridx, bcol])
                plsc.store_scatter(rows_v, [ridx, bcol], va * vb)
                return carry

            jax.lax.fori_loop(0, _W // 2, mul, 0)
            pltpu.sync_copy(rows_v, acc_sh.at[iji_v], add=True)
            return _

        jax.lax.fori_loop(0, _TPT // _W, win, 0)
        plsc.subcore_barrier()
        pltpu.sync_copy(acc_sh.at[pl.ds(sid * _EPT, _EPT)],
                        out_ref.at[chunk].at[pl.ds(sid * _EPT, _EPT)])
        plsc.subcore_barrier()


_TRIP_CACHE = {}


def _sc_trip(xd_c, s_c, ikj, iji, zeros8):
    if "k" not in _TRIP_CACHE:
        mesh = plsc.VectorSubcoreMesh(core_axis_name="c", subcore_axis_name="s")
        _TRIP_CACHE["k"] = functools.partial(
            pl.kernel,
            mesh=mesh,
            compiler_params=pltpu.CompilerParams(use_tc_tiling_on_sc=False,
                                                 needs_layout_passes=False),
            out_type=jax.ShapeDtypeStruct((8, _NE, 8), jnp.float32),
            scratch_types=[
                pltpu.VMEM((_W,), jnp.int32),
                pltpu.VMEM((_W,), jnp.int32),
                pltpu.VMEM((_W, 8), jnp.float32),
                pltpu.VMEM((_W, 8), jnp.float32),
                pltpu.VMEM((_W, 8), jnp.float32),
                pltpu.VMEM_SHARED((_NE, 8), jnp.float32),
                pltpu.SemaphoreType.DMA,
            ],
        )(_trip_body)
    return _TRIP_CACHE["k"](xd_c, s_c, ikj, iji, zeros8)


def kernel(params, z, cond, rbf, sbf, i, j, idx_kj, idx_ji):
    w1 = params["emb_lin_w"][: _HC + _COND]
    w2 = params["emb_lin_w"][_HC + _COND: 2 * (_HC + _COND)]
    w3 = params["emb_lin_w"][2 * (_HC + _COND):]

    ha, hb = _node_embed(z.astype(jnp.int32).reshape(_NN, 1), cond, params["emb"], w1, w2)

    ga = jnp.take(ha, i, axis=0)
    gb = jnp.take(hb, j, axis=0)
    x, g0 = _edge_init(ga, gb, rbf, params["emb_rbf_w"], params["emb_rbf_b"],
                       w3, params["emb_lin_b"], params["outs"][0]["rbf"])

    sbf1_all = jnp.concatenate([b["sbf1"] for b in params["blocks"]], axis=1)
    t_all = _mm(sbf, sbf1_all, _TRIP_BLK)

    ikj = idx_kj.astype(jnp.int32)
    iji = idx_ji.astype(jnp.int32)
    zeros8 = jnp.zeros((_W, 8), jnp.float32)

    gs = [jax.ops.segment_sum(g0, i, num_segments=_NN)]
    for bi in range(_NB):
        b = params["blocks"][bi]
        xji, xdown = _pre(x, rbf, b)
        s64 = _mm(t_all[:, bi * _BE:(bi + 1) * _BE], b["sbf2"], _TRIP_BLK)
        xd_c = jnp.transpose(xdown.reshape(_NE, 8, 8), (1, 0, 2))
        s_c = jnp.transpose(s64.reshape(_NT, 8, 8), (1, 0, 2))
        acc8 = _sc_trip(xd_c, s_c, ikj, iji, zeros8)
        acc = jnp.transpose(acc8, (1, 0, 2)).reshape(_NE, _IE)
        x, g = _post(acc, xji, x, rbf, b, params["outs"][bi + 1]["rbf"])
        gs.append(jax.ops.segment_sum(g, i, num_segments=_NN))

    upw = jnp.stack([o["up_w"] for o in params["outs"]])
    upb = jnp.stack([o["up_b"] for o in params["outs"]])
    lw = jnp.stack([jnp.stack([l["w"] for l in o["lins"]]) for o in params["outs"]])
    lb = jnp.stack([jnp.stack([l["b"] for l in o["lins"]]) for o in params["outs"]])
    linw = jnp.stack([o["lin"] for o in params["outs"]])
    return _out_mlps(jnp.stack(gs), upw, upb, lw, lb, linw)
